# baseline (device time: 244276 ns/iter reference)
import jax
import jax.numpy as jnp
from jax import lax
from jax.experimental import pallas as pl
from jax.experimental.pallas import tpu as pltpu

CH = 128
W = 128


def kernel(x, dest):
    T, D = x.shape
    my_x = lax.axis_index("x")

    x3 = jnp.reshape(x, (T, 8, 128))

    n_keep = jnp.sum((dest == my_x).astype(jnp.int32))
    n_send = T - n_keep

    keep_base = jnp.where(my_x == 0, 0, n_send)
    send_base = jnp.where(my_x == 1, n_keep, 0)
    scalars = jnp.stack([n_keep, n_send, keep_base, send_base]).astype(jnp.int32)

    def body(scal_ref, dest_ref, x_ref, out_ref, stage, local_sem, stage_sems, send_sem, recv_sem):
        mx = lax.axis_index("x")
        my_ = lax.axis_index("y")
        mz = lax.axis_index("z")
        peer = (1 - mx, my_, mz)

        barrier = pltpu.get_barrier_semaphore()
        pl.semaphore_signal(
            barrier, inc=1, device_id=peer, device_id_type=pl.DeviceIdType.MESH
        )
        pl.semaphore_wait(barrier, 1)

        nk = scal_ref[0]
        ns = scal_ref[1]
        kb = scal_ref[2]
        sb = scal_ref[3]

        def chunk_rdma(src_off, dst_off):
            return pltpu.make_async_remote_copy(
                src_ref=stage.at[pl.ds(src_off, CH)],
                dst_ref=out_ref.at[pl.ds(dst_off, CH)],
                send_sem=send_sem,
                recv_sem=recv_sem,
                device_id=peer,
                device_id_type=pl.DeviceIdType.MESH,
            )

        def wait_stage(rows, c):
            pltpu.make_async_copy(
                x_ref.at[pl.ds(0, rows)],
                stage.at[pl.ds(0, rows)],
                stage_sems.at[c],
            ).wait()

        def it(i, carry):
            cs, ck = carry
            is_keep = dest_ref[i] == mx
            src = x_ref.at[pl.ds(i, 1)]

            @pl.when(is_keep)
            def _():
                pltpu.make_async_copy(
                    src, out_ref.at[pl.ds(kb + ck, 1)], local_sem
                ).start()

            @pl.when(jnp.logical_not(is_keep))
            def _():
                pltpu.make_async_copy(
                    src, stage.at[pl.ds(cs, 1)], stage_sems.at[cs // CH]
                ).start()

                @pl.when((cs + 1) % CH == 0)
                def _():
                    c_prev = (cs + 1) // CH - 2

                    @pl.when(c_prev >= 0)
                    def _():
                        wait_stage(CH, c_prev)
                        chunk_rdma(CH * c_prev, sb + CH * c_prev).start()

            ik = is_keep.astype(jnp.int32)
            return (cs + 1 - ik, ck + ik)

        lax.fori_loop(0, T, it, (jnp.int32(0), jnp.int32(0)))

        nchunks = ns // CH
        rem = ns % CH
        nchunks_total = nchunks + jnp.where(rem > 0, 1, 0)

        @pl.when(nchunks >= 1)
        def _():
            wait_stage(CH, nchunks - 1)
            chunk_rdma(CH * (nchunks - 1), sb + CH * (nchunks - 1)).start()

        @pl.when(rem > 0)
        def _():
            def w1(i, c):
                wait_stage(1, nchunks)
                return c

            lax.fori_loop(0, rem, w1, 0)
            rb = jnp.maximum(ns - CH, 0)
            chunk_rdma(rb, sb + rb).start()

        def wl(i, c):
            pltpu.make_async_copy(
                x_ref.at[pl.ds(0, W)], out_ref.at[pl.ds(0, W)], local_sem
            ).wait()
            return c

        lax.fori_loop(0, nk // W, wl, 0)

        def wl1(i, c):
            pltpu.make_async_copy(
                x_ref.at[pl.ds(0, 1)], out_ref.at[pl.ds(0, 1)], local_sem
            ).wait()
            return c

        lax.fori_loop(0, nk % W, wl1, 0)

        def ws(i, c):
            chunk_rdma(0, 0).wait_send()
            return c

        lax.fori_loop(0, nchunks_total, ws, 0)

        def wr(i, c):
            chunk_rdma(0, 0).wait_recv()
            return c

        lax.fori_loop(0, nchunks_total, wr, 0)

    out3 = pl.pallas_call(
        body,
        out_shape=jax.ShapeDtypeStruct((T, 8, 128), x.dtype),
        in_specs=[
            pl.BlockSpec(memory_space=pltpu.SMEM),
            pl.BlockSpec(memory_space=pltpu.SMEM),
            pl.BlockSpec(memory_space=pltpu.MemorySpace.HBM),
        ],
        out_specs=pl.BlockSpec(memory_space=pltpu.MemorySpace.HBM),
        scratch_shapes=[
            pltpu.VMEM((T, 8, 128), jnp.float32),
            pltpu.SemaphoreType.DMA,
            pltpu.SemaphoreType.DMA((T // CH,)),
            pltpu.SemaphoreType.DMA,
            pltpu.SemaphoreType.DMA,
        ],
        compiler_params=pltpu.CompilerParams(collective_id=0),
    )(scalars, dest.astype(jnp.int32), x3)
    return jnp.reshape(out3, (T, D))


# device time: 126610 ns/iter; 1.9294x vs baseline; 1.9294x over previous
import jax
import jax.numpy as jnp
from jax import lax
from jax.experimental import pallas as pl
from jax.experimental.pallas import tpu as pltpu

W = 128


def kernel(x, dest):
    T, D = x.shape
    my_x = lax.axis_index("x")

    x3 = jnp.reshape(x, (T, 8, 128))

    n_keep = jnp.sum((dest == my_x).astype(jnp.int32))
    n_send = T - n_keep

    keep_base = jnp.where(my_x == 0, 0, n_send)
    send_base = jnp.where(my_x == 1, n_keep, 0)
    scalars = jnp.stack([n_keep, n_send, keep_base, send_base]).astype(jnp.int32)

    def body(scal_ref, dest_ref, x_ref, out_ref, local_sem, send_sem, recv_sem):
        mx = lax.axis_index("x")
        my_ = lax.axis_index("y")
        mz = lax.axis_index("z")
        peer = (1 - mx, my_, mz)

        barrier = pltpu.get_barrier_semaphore()
        pl.semaphore_signal(
            barrier, inc=1, device_id=peer, device_id_type=pl.DeviceIdType.MESH
        )
        pl.semaphore_wait(barrier, 1)

        nk = scal_ref[0]
        ns = scal_ref[1]
        kb = scal_ref[2]
        sb = scal_ref[3]

        def local_copy(src_i, dst_i, rows):
            pltpu.make_async_copy(
                x_ref.at[pl.ds(src_i, rows)],
                out_ref.at[pl.ds(dst_i, rows)],
                local_sem,
            ).start()

        def remote_copy(src_i, dst_i, rows):
            pltpu.make_async_remote_copy(
                src_ref=x_ref.at[pl.ds(src_i, rows)],
                dst_ref=out_ref.at[pl.ds(dst_i, rows)],
                send_sem=send_sem,
                recv_sem=recv_sem,
                device_id=peer,
                device_id_type=pl.DeviceIdType.MESH,
            ).start()

        def it(p, carry):
            cs, ck = carry
            i = 2 * p
            k0 = dest_ref[i] == mx
            k1 = dest_ref[i + 1] == mx

            @pl.when(k0 & k1)
            def _():
                local_copy(i, kb + ck, 2)

            @pl.when(jnp.logical_not(k0) & jnp.logical_not(k1))
            def _():
                remote_copy(i, sb + cs, 2)

            @pl.when(k0 != k1)
            def _():
                ki = jnp.where(k0, i, i + 1)
                si = jnp.where(k0, i + 1, i)
                local_copy(ki, kb + ck, 1)
                remote_copy(si, sb + cs, 1)

            nki = k0.astype(jnp.int32) + k1.astype(jnp.int32)
            return (cs + 2 - nki, ck + nki)

        lax.fori_loop(0, T // 2, it, (jnp.int32(0), jnp.int32(0)))

        def mk_remote(rows):
            return pltpu.make_async_remote_copy(
                src_ref=x_ref.at[pl.ds(0, rows)],
                dst_ref=out_ref.at[pl.ds(0, rows)],
                send_sem=send_sem,
                recv_sem=recv_sem,
                device_id=peer,
                device_id_type=pl.DeviceIdType.MESH,
            )

        def mk_local(rows):
            return pltpu.make_async_copy(
                x_ref.at[pl.ds(0, rows)], out_ref.at[pl.ds(0, rows)], local_sem
            )

        def drain(n, wait_batch, wait_one):
            def wb(i, c):
                wait_batch()
                return c

            lax.fori_loop(0, n // W, wb, 0)

            def w1(i, c):
                wait_one()
                return c

            lax.fori_loop(0, n % W, w1, 0)

        drain(nk, lambda: mk_local(W).wait(), lambda: mk_local(1).wait())
        drain(ns, lambda: mk_remote(W).wait_send(), lambda: mk_remote(1).wait_send())
        drain(ns, lambda: mk_remote(W).wait_recv(), lambda: mk_remote(1).wait_recv())

    out3 = pl.pallas_call(
        body,
        out_shape=jax.ShapeDtypeStruct((T, 8, 128), x.dtype),
        in_specs=[
            pl.BlockSpec(memory_space=pltpu.SMEM),
            pl.BlockSpec(memory_space=pltpu.SMEM),
            pl.BlockSpec(memory_space=pltpu.MemorySpace.HBM),
        ],
        out_specs=pl.BlockSpec(memory_space=pltpu.MemorySpace.HBM),
        scratch_shapes=[
            pltpu.SemaphoreType.DMA,
            pltpu.SemaphoreType.DMA,
            pltpu.SemaphoreType.DMA,
        ],
        compiler_params=pltpu.CompilerParams(collective_id=0),
    )(scalars, dest.astype(jnp.int32), x3)
    return jnp.reshape(out3, (T, D))
